# baseline (speedup 1.0000x reference)
import functools

import jax
import jax.numpy as jnp
from jax import lax
from jax.experimental import pallas as pl
from jax.experimental.pallas import tpu as pltpu
from jax.experimental.pallas import tpu_sc as plsc

N = 1024
M = 100000
B = 16384

NC = 2
NS = 16
LANES = 16
NW = NC * NS
ROWS_PER_W = N // NW
OUT_CHUNK = 8192


def _sc_gather(e_hbm, idx_hbm, out_hbm, row_v, idx_v, out_v):
    wid = lax.axis_index("s") * NC + lax.axis_index("c")
    pltpu.sync_copy(idx_hbm, idx_v)

    @pl.loop(0, ROWS_PER_W)
    def _row(r):
        n = wid * ROWS_PER_W + r
        pltpu.sync_copy(e_hbm.at[n], row_v)
        for h in range(B // OUT_CHUNK):
            @pl.loop(0, OUT_CHUNK // LANES, unroll=8)
            def _gather(j):
                iv = idx_v[pl.ds(h * OUT_CHUNK + j * LANES, LANES)]
                out_v[pl.ds(j * LANES, LANES)] = plsc.load_gather(row_v, [iv])
            pltpu.sync_copy(out_v, out_hbm.at[n, pl.ds(h * OUT_CHUNK, OUT_CHUNK)])


def _sc_gather_call(e, idx):
    mesh = plsc.VectorSubcoreMesh(core_axis_name="c", subcore_axis_name="s")
    return pl.kernel(
        _sc_gather,
        out_type=jax.ShapeDtypeStruct((N, B), jnp.float32),
        mesh=mesh,
        compiler_params=pltpu.CompilerParams(needs_layout_passes=False),
        scratch_types=[
            pltpu.VMEM((M,), jnp.float32),
            pltpu.VMEM((B,), jnp.int32),
            pltpu.VMEM((OUT_CHUNK,), jnp.float32),
        ],
    )(e, idx)


LSE_TM = 512
LSE_GRID = (M + LSE_TM - 1) // LSE_TM


def _lse_kernel(e_ref, o_ref, m_scr, s_scr):
    i = pl.program_id(0)
    blk = e_ref[...]
    cols = i * LSE_TM + lax.broadcasted_iota(jnp.int32, blk.shape, 1)
    blk = jnp.where(cols < M, blk, -jnp.inf)
    bm = jnp.max(blk, axis=1, keepdims=True)
    bs = jnp.sum(jnp.exp(blk - bm), axis=1, keepdims=True)

    @pl.when(i == 0)
    def _():
        m_scr[...] = bm
        s_scr[...] = bs

    @pl.when(i > 0)
    def _():
        m_old = m_scr[...]
        s_old = s_scr[...]
        m_new = jnp.maximum(m_old, bm)
        s_scr[...] = s_old * jnp.exp(m_old - m_new) + bs * jnp.exp(bm - m_new)
        m_scr[...] = m_new

    @pl.when(i == LSE_GRID - 1)
    def _():
        o_ref[...] = m_scr[...] + jnp.log(s_scr[...])


def _lse_call(e):
    return pl.pallas_call(
        _lse_kernel,
        grid=(LSE_GRID,),
        in_specs=[pl.BlockSpec((N, LSE_TM), lambda i: (0, i))],
        out_specs=pl.BlockSpec((N, 1), lambda i: (0, 0)),
        out_shape=jax.ShapeDtypeStruct((N, 1), jnp.float32),
        scratch_shapes=[
            pltpu.VMEM((N, 1), jnp.float32),
            pltpu.VMEM((N, 1), jnp.float32),
        ],
    )(e)


TS_TN = 512
TS_TB = 512


def _transsub_kernel(g_ref, lse_ref, o_ref):
    o_ref[...] = (g_ref[...] - lse_ref[...]).T


def _transsub_call(g, lse):
    return pl.pallas_call(
        _transsub_kernel,
        grid=(B // TS_TB, N // TS_TN),
        in_specs=[
            pl.BlockSpec((TS_TN, TS_TB), lambda i, j: (j, i)),
            pl.BlockSpec((TS_TN, 1), lambda i, j: (j, 0)),
        ],
        out_specs=pl.BlockSpec((TS_TB, TS_TN), lambda i, j: (i, j)),
        out_shape=jax.ShapeDtypeStruct((B, N), jnp.float32),
    )(g, lse)


@jax.jit
def kernel(x_t, unnormalized_emission_matrix):
    idx = x_t.astype(jnp.int32)
    e = unnormalized_emission_matrix
    lse = _lse_call(e)
    gathered = _sc_gather_call(e, idx)
    return _transsub_call(gathered, lse)
